# Initial kernel scaffold; baseline (speedup 1.0000x reference)
import jax
import jax.numpy as jnp
from jax.experimental import pallas as pl


def kernel(X, edge_index, W_l, W_r, b):
    raise NotImplementedError("write your pallas kernel here")


# speedup vs baseline: 5.4516x; 5.4516x over previous
import jax
import jax.numpy as jnp
from jax import lax
from jax.experimental import pallas as pl
from jax.experimental.pallas import tpu as pltpu
from jax.experimental.pallas import tpu_sc as plsc

NC = 2
NS = 16
K = 80


def _sc_pass(table, gidx, sidx, zeros_w, cidx=None, zeros_c=None,
             ones_c=None, *, n, w):
    R = gidx.shape[0] // NC
    C = R // NS
    npt = n // NS

    def body(*refs):
        if cidx is None:
            (table_r, gidx_r, sidx_r, zeros_r, out_r,
             gbuf, sbuf, rows, acc, sem) = refs
        else:
            (table_r, gidx_r, sidx_r, zeros_r, cidx_r, zc_r, ones_r,
             out_r, cout_r, gbuf, sbuf, rows, acc, cbuf, ones_v, cacc,
             sem) = refs
        c = lax.axis_index("c")
        s = lax.axis_index("s")
        pltpu.sync_copy(zeros_r.at[pl.ds(s * npt, npt)],
                        acc.at[pl.ds(s * npt, npt)])
        pltpu.sync_copy(gidx_r.at[pl.ds(c * R + s * C, C)], gbuf)
        pltpu.sync_copy(sidx_r.at[pl.ds(s * C, C)], sbuf)
        if cidx is not None:
            pltpu.sync_copy(zc_r.at[pl.ds(s * npt, npt)],
                            cacc.at[pl.ds(s * npt, npt)])
            pltpu.sync_copy(cidx_r.at[pl.ds(c * R + s * C, C)], cbuf)
            pltpu.sync_copy(ones_r, ones_v)
        plsc.subcore_barrier()

        def chunk(j, carry):
            pltpu.async_copy(table_r.at[gbuf.at[j]], rows, sem).wait()
            pltpu.sync_copy(rows, acc.at[sbuf.at[j]], add=True)
            if cidx is not None:
                pltpu.sync_copy(ones_v, cacc.at[cbuf.at[j]], add=True)
            return carry

        lax.fori_loop(0, C, chunk, 0)
        plsc.subcore_barrier()
        pltpu.sync_copy(acc.at[pl.ds(s * npt, npt)],
                        out_r.at[pl.ds(c * n + s * npt, npt)])
        if cidx is not None:
            pltpu.sync_copy(cacc.at[pl.ds(s * npt, npt)],
                            cout_r.at[pl.ds(c * n + s * npt, npt)])

    out_type = [jax.ShapeDtypeStruct((2 * n, w), jnp.float32)]
    scratch = [
        pltpu.VMEM((C, K), jnp.int32),
        pltpu.VMEM((C, K), jnp.int32),
        pltpu.VMEM((K, w), jnp.float32),
        pltpu.VMEM_SHARED((n, w), jnp.float32),
    ]
    args = [table, gidx, sidx, zeros_w]
    if cidx is not None:
        out_type.append(jax.ShapeDtypeStruct((2 * n, 16), jnp.float32))
        scratch += [
            pltpu.VMEM((C, K), jnp.int32),
            pltpu.VMEM((K, 16), jnp.float32),
            pltpu.VMEM_SHARED((n, 16), jnp.float32),
        ]
        args += [cidx, zeros_c, ones_c]
    scratch.append(pltpu.SemaphoreType.DMA)

    f = pl.kernel(
        body,
        out_type=tuple(out_type),
        mesh=plsc.VectorSubcoreMesh(core_axis_name="c", subcore_axis_name="s"),
        scratch_types=scratch,
        compiler_params=pltpu.CompilerParams(use_tc_tiling_on_sc=False),
    )
    res = f(*args)
    return res


def _tc_dense_body(sum_ref, cnt_ref, x_ref, wl_ref, wr_ref, b_ref, g_ref):
    agg = jnp.concatenate([sum_ref[0], sum_ref[1]], axis=1)
    cnt = cnt_ref[0, :, 0:1]
    mean = agg / jnp.maximum(cnt, 1.0)
    h = jnp.dot(mean, wl_ref[:], preferred_element_type=jnp.float32)
    h += jnp.dot(x_ref[:], wr_ref[:], preferred_element_type=jnp.float32)
    h = jnp.maximum(h + b_ref[:], 0.0)
    g_ref[0] = h
    g_ref[1] = h * h


def _tc_final_body(acc_ref, cnt_ref, g_ref, gg_ref):
    s1 = acc_ref[0]
    s2 = acc_ref[1]
    scnt = cnt_ref[1, :, 0:1]
    h = g_ref[0]
    h2 = g_ref[1]
    num = scnt * h2 - 2.0 * h * s1 + s2
    gg_ref[:] = jnp.tanh(num / jnp.maximum(scnt, 1.0))


def kernel(X, edge_index, W_l, W_r, b):
    N, D = X.shape
    E = edge_index.shape[1]
    assert D == 128 and E % (K * NC * NS) == 0 and N % NS == 0

    src = edge_index[0]
    dst = edge_index[1]
    src2d = src.reshape(E // K, K)
    dst2d = dst.reshape(E // K, K)
    srcx2d = jnp.concatenate([src2d, src2d + N], axis=0)
    dstx2d = jnp.concatenate([dst2d, dst2d + N], axis=0)
    cidx2d = jnp.concatenate([dst2d, src2d], axis=0)

    xcols = jnp.concatenate([X[:, :64], X[:, 64:]], axis=0)
    zeros64 = jnp.zeros((N, 64), jnp.float32)
    zeros16 = jnp.zeros((N, 16), jnp.float32)
    zeros128 = jnp.zeros((N, 128), jnp.float32)
    ones16 = jnp.ones((K, 16), jnp.float32)

    sums, cnts = _sc_pass(xcols, srcx2d, dst2d, zeros64,
                          cidx2d, zeros16, ones16, n=N, w=64)

    R = 1000
    grid = (N // R,)
    g = pl.pallas_call(
        _tc_dense_body,
        grid=grid,
        in_specs=[
            pl.BlockSpec((2, R, 64), lambda i: (0, i, 0)),
            pl.BlockSpec((2, R, 16), lambda i: (0, i, 0)),
            pl.BlockSpec((R, D), lambda i: (i, 0)),
            pl.BlockSpec((D, D), lambda i: (0, 0)),
            pl.BlockSpec((D, D), lambda i: (0, 0)),
            pl.BlockSpec((1, D), lambda i: (0, 0)),
        ],
        out_specs=pl.BlockSpec((2, R, D), lambda i: (0, i, 0)),
        out_shape=jax.ShapeDtypeStruct((2, N, D), jnp.float32),
    )(sums.reshape(2, N, 64), cnts.reshape(2, N, 16), X, W_l, W_r,
      b.reshape(1, D))

    (acc3,) = _sc_pass(g.reshape(2 * N, D), dstx2d, src2d, zeros128,
                       n=N, w=D)

    gg = pl.pallas_call(
        _tc_final_body,
        grid=grid,
        in_specs=[
            pl.BlockSpec((2, R, D), lambda i: (0, i, 0)),
            pl.BlockSpec((2, R, 16), lambda i: (0, i, 0)),
            pl.BlockSpec((2, R, D), lambda i: (0, i, 0)),
        ],
        out_specs=pl.BlockSpec((R, D), lambda i: (i, 0)),
        out_shape=jax.ShapeDtypeStruct((N, D), jnp.float32),
    )(acc3.reshape(2, N, D), cnts.reshape(2, N, 16), g)
    return gg


# speedup vs baseline: 8.9927x; 1.6496x over previous
import jax
import jax.numpy as jnp
from jax import lax
from jax.experimental import pallas as pl
from jax.experimental.pallas import tpu as pltpu
from jax.experimental.pallas import tpu_sc as plsc

NC = 2
NS = 16
K = 80


KP = (K + 31) // 32 * 16


def _unpack_chunk(buf, j, stage):
    vecs = []
    for blk in range(KP // 16):
        word = buf[j, pl.ds(blk * 16, 16)]
        lo = word & 0xFFFF
        hi = word >> 16
        stage[pl.ds(blk * 32, 16)] = lo
        stage[pl.ds(blk * 32 + 16, 16)] = hi
        vecs += [lo, hi]
    return vecs[:K // 16]


def _sc_pass(table, gidx, sidx, zeros_w, zeros_n=None, *, n, w, counts):
    R = gidx.shape[0] // NC
    C = R // NS
    npt = n // NS
    KS = 2 * KP

    def body(*refs):
        if counts:
            (table_r, gidx_r, sidx_r, zeros_r, zn_r, out_r, hout_r,
             gbuf, sbuf, rows0, rows1, gi0, gi1, si, acc, hist,
             sem0, sem1) = refs
        else:
            (table_r, gidx_r, sidx_r, zeros_r, out_r,
             gbuf, sbuf, rows0, rows1, gi0, gi1, si, acc,
             sem0, sem1) = refs
        c = lax.axis_index("c")
        s = lax.axis_index("s")
        pltpu.sync_copy(zeros_r.at[pl.ds(s * npt, npt)],
                        acc.at[pl.ds(s * npt, npt)])
        pltpu.sync_copy(gidx_r.at[pl.ds(c * R + s * C, C)], gbuf)
        pltpu.sync_copy(sidx_r.at[pl.ds(s * C, C)], sbuf)
        if counts:
            pltpu.sync_copy(zn_r, hist)
        plsc.subcore_barrier()

        _unpack_chunk(gbuf, 0, gi0)
        pltpu.async_copy(table_r.at[gi0.at[pl.ds(0, K)]], rows0, sem0)
        _unpack_chunk(gbuf, 1, gi1)
        pltpu.async_copy(table_r.at[gi1.at[pl.ds(0, K)]], rows1, sem1)

        @pl.loop(0, C, step=2)
        def _(i):
            for off, rows, gi, sem in ((0, rows0, gi0, sem0),
                                       (1, rows1, gi1, sem1)):
                j = i + off
                pltpu.make_async_copy(table_r.at[gi.at[pl.ds(0, K)]], rows,
                                      sem).wait()
                svecs = _unpack_chunk(sbuf, j, si)
                pltpu.sync_copy(rows, acc.at[si.at[pl.ds(0, K)]], add=True)
                if counts:
                    ones = jnp.ones((16,), jnp.float32)
                    for t, a in enumerate(svecs):
                        bvec = gi[pl.ds(t * 16, 16)] - n
                        v = jnp.where(c == 0, a, bvec)
                        plsc.addupdate_scatter(hist, [v], ones)

                @pl.when(j + 2 < C)
                def _issue():
                    _unpack_chunk(gbuf, j + 2, gi)
                    pltpu.async_copy(table_r.at[gi.at[pl.ds(0, K)]], rows,
                                    sem)

        plsc.subcore_barrier()
        pltpu.sync_copy(acc.at[pl.ds(s * npt, npt)],
                        out_r.at[pl.ds(c * n + s * npt, npt)])
        if counts:
            pltpu.sync_copy(hist, hout_r.at[c * NS + s])

    out_type = [jax.ShapeDtypeStruct((2 * n, w), jnp.float32)]
    scratch = [
        pltpu.VMEM((C, KP), jnp.int32),
        pltpu.VMEM((C, KP), jnp.int32),
        pltpu.VMEM((K, w), jnp.float32),
        pltpu.VMEM((K, w), jnp.float32),
        pltpu.VMEM((KS,), jnp.int32),
        pltpu.VMEM((KS,), jnp.int32),
        pltpu.VMEM((KS,), jnp.int32),
        pltpu.VMEM_SHARED((n, w), jnp.float32),
    ]
    args = [table, gidx, sidx, zeros_w]
    if counts:
        out_type.append(jax.ShapeDtypeStruct((2 * NS, n), jnp.float32))
        scratch.append(pltpu.VMEM((n,), jnp.float32))
        args.append(zeros_n)
    scratch += [pltpu.SemaphoreType.DMA, pltpu.SemaphoreType.DMA]

    f = pl.kernel(
        body,
        out_type=tuple(out_type),
        mesh=plsc.VectorSubcoreMesh(core_axis_name="c", subcore_axis_name="s"),
        scratch_types=scratch,
        compiler_params=pltpu.CompilerParams(use_tc_tiling_on_sc=False,
                                             needs_layout_passes=False),
    )
    return f(*args)


def _pack_idx(idx2d):
    rows = idx2d.shape[0]
    padded = jnp.concatenate(
        [idx2d, jnp.zeros((rows, 16), jnp.int32)], axis=1)
    quads = padded.reshape(rows, 3, 2, 16)
    return (quads[:, :, 0, :] | (quads[:, :, 1, :] << 16)).reshape(rows, KP)


def _tc_reduce_body(hist_ref, cnt_ref):
    cnt_ref[...] = jnp.sum(hist_ref[...], axis=1)[..., None]


def _tc_dense_body(sum_ref, cnt_ref, x_ref, wl_ref, wr_ref, b_ref, g_ref):
    agg = jnp.concatenate([sum_ref[0], sum_ref[1]], axis=1)
    cnt = cnt_ref[0]
    mean = agg / jnp.maximum(cnt, 1.0)
    h = jnp.dot(mean, wl_ref[:], preferred_element_type=jnp.float32)
    h += jnp.dot(x_ref[:], wr_ref[:], preferred_element_type=jnp.float32)
    h = jnp.maximum(h + b_ref[:], 0.0)
    g_ref[0] = h
    g_ref[1] = h * h


def _tc_final_body(acc_ref, cnt_ref, g_ref, gg_ref):
    s1 = acc_ref[0]
    s2 = acc_ref[1]
    scnt = cnt_ref[0]
    h = g_ref[0]
    h2 = g_ref[1]
    num = scnt * h2 - 2.0 * h * s1 + s2
    gg_ref[:] = jnp.tanh(num / jnp.maximum(scnt, 1.0))


def kernel(X, edge_index, W_l, W_r, b):
    N, D = X.shape
    E = edge_index.shape[1]
    assert D == 128 and E % (K * NC * NS) == 0 and N % NS == 0

    src = edge_index[0]
    dst = edge_index[1]
    src2d = src.reshape(E // K, K)
    dst2d = dst.reshape(E // K, K)
    srcx2d = _pack_idx(jnp.concatenate([src2d, src2d + N], axis=0))
    dstx2d = _pack_idx(jnp.concatenate([dst2d, dst2d + N], axis=0))
    src2d = _pack_idx(src2d)
    dst2d = _pack_idx(dst2d)

    xcols = jnp.concatenate([X[:, :64], X[:, 64:]], axis=0)
    zeros64 = jnp.zeros((N, 64), jnp.float32)
    zeros128 = jnp.zeros((N, 128), jnp.float32)
    zeros_n = jnp.zeros((N,), jnp.float32)

    sums, hists = _sc_pass(xcols, srcx2d, dst2d, zeros64, zeros_n,
                           n=N, w=64, counts=True)

    cnts = pl.pallas_call(
        _tc_reduce_body,
        out_shape=jax.ShapeDtypeStruct((2, N, 1), jnp.float32),
    )(hists.reshape(2, NS, N))

    R = 1000
    grid = (N // R,)
    g = pl.pallas_call(
        _tc_dense_body,
        grid=grid,
        in_specs=[
            pl.BlockSpec((2, R, 64), lambda i: (0, i, 0)),
            pl.BlockSpec((1, R, 1), lambda i: (0, i, 0)),
            pl.BlockSpec((R, D), lambda i: (i, 0)),
            pl.BlockSpec((D, D), lambda i: (0, 0)),
            pl.BlockSpec((D, D), lambda i: (0, 0)),
            pl.BlockSpec((1, D), lambda i: (0, 0)),
        ],
        out_specs=pl.BlockSpec((2, R, D), lambda i: (0, i, 0)),
        out_shape=jax.ShapeDtypeStruct((2, N, D), jnp.float32),
    )(sums.reshape(2, N, 64), cnts, X, W_l, W_r, b.reshape(1, D))

    (acc3,) = _sc_pass(g.reshape(2 * N, D), dstx2d, src2d, zeros128,
                       n=N, w=D, counts=False)

    gg = pl.pallas_call(
        _tc_final_body,
        grid=grid,
        in_specs=[
            pl.BlockSpec((2, R, D), lambda i: (0, i, 0)),
            pl.BlockSpec((1, R, 1), lambda i: (1, i, 0)),
            pl.BlockSpec((2, R, D), lambda i: (0, i, 0)),
        ],
        out_specs=pl.BlockSpec((R, D), lambda i: (i, 0)),
        out_shape=jax.ShapeDtypeStruct((N, D), jnp.float32),
    )(acc3.reshape(2, N, D), cnts, g)
    return gg


# speedup vs baseline: 10.3763x; 1.1539x over previous
import jax
import jax.numpy as jnp
from jax import lax
from jax.experimental import pallas as pl
from jax.experimental.pallas import tpu as pltpu
from jax.experimental.pallas import tpu_sc as plsc

NC = 2
NS = 16
K = 80


KP = (K + 31) // 32 * 16


def _unpack_chunk(buf, j, stage):
    vecs = []
    for blk in range(KP // 16):
        word = buf[j, pl.ds(blk * 16, 16)]
        lo = word & 0xFFFF
        hi = word >> 16
        stage[pl.ds(blk * 32, 16)] = lo
        stage[pl.ds(blk * 32 + 16, 16)] = hi
        vecs += [lo, hi]
    return vecs[:K // 16]


def _sc_pass(table, gidx, sidx, zeros_w, zeros_n=None, *, n, w, counts,
             nbuf=3, nphase=1):
    R = gidx.shape[0] // NC
    C = R // NS
    CP = C // nphase
    npt = n // NS
    KS = 2 * KP
    assert C % nphase == 0

    NB = nbuf

    def body(*refs):
        if counts:
            (table_r, gidx_r, sidx_r, zeros_r, zn_r, out_r, hout_r,
             gbuf, sbuf, *rest) = refs
        else:
            (table_r, gidx_r, sidx_r, zeros_r, out_r,
             gbuf, sbuf, *rest) = refs
        rows = rest[0:NB]
        gi = rest[NB:2 * NB]
        si = rest[2 * NB:3 * NB]
        acc = rest[3 * NB]
        p = 3 * NB + 1
        if counts:
            hist = rest[p]
            p += 1
        gsem = rest[p:p + NB]
        ssem = rest[p + NB:p + 2 * NB]
        c = lax.axis_index("c")
        s = lax.axis_index("s")
        pltpu.sync_copy(zeros_r.at[pl.ds(s * npt, npt)],
                        acc.at[pl.ds(s * npt, npt)])
        if counts:
            pltpu.sync_copy(zn_r, hist)
        plsc.subcore_barrier()

        def gather_descr(b):
            return pltpu.make_async_copy(
                table_r.at[gi[b].at[pl.ds(0, K)]], rows[b], gsem[b])

        def slot(j, b):
            gather_descr(b).wait()
            svecs = _unpack_chunk(sbuf, j, si[b])
            pltpu.async_copy(rows[b], acc.at[si[b].at[pl.ds(0, K)]],
                             ssem[b], add=True)
            if counts:
                ones = jnp.ones((16,), jnp.float32)
                for t, a in enumerate(svecs):
                    bvec = gi[b][pl.ds(t * 16, 16)] - n
                    v = jnp.where(c == 0, a, bvec)
                    plsc.addupdate_scatter(hist, [v], ones)
            bp = (b + 2) % NB

            def _drain():
                pltpu.make_async_copy(
                    rows[bp], acc.at[si[bp].at[pl.ds(0, K)]],
                    ssem[bp]).wait()

            if NB == 2:
                _drain()
            else:
                pl.when(j + 2 >= NB)(_drain)

            @pl.when(j + 2 < CP)
            def _issue():
                _unpack_chunk(gbuf, j + 2, gi[bp])
                pltpu.async_copy(table_r.at[gi[bp].at[pl.ds(0, K)]],
                                 rows[bp], gsem[bp])

        CB = CP - CP % NB
        for ph in range(nphase):
            pltpu.sync_copy(
                gidx_r.at[pl.ds(c * R + s * C + ph * CP, CP)], gbuf)
            pltpu.sync_copy(sidx_r.at[pl.ds(s * C + ph * CP, CP)], sbuf)
            for b in range(2):
                _unpack_chunk(gbuf, b, gi[b])
                pltpu.async_copy(table_r.at[gi[b].at[pl.ds(0, K)]],
                                 rows[b], gsem[b])

            @pl.loop(0, CB, step=NB)
            def _(i):
                for off in range(NB):
                    slot(i + off, off)

            for j in range(CB, CP):
                slot(jnp.int32(j), j % NB)
            for j in range(CP + 2 - NB, CP):
                b_last = j % NB
                pltpu.make_async_copy(rows[b_last],
                                      acc.at[si[b_last].at[pl.ds(0, K)]],
                                      ssem[b_last]).wait()

        plsc.subcore_barrier()
        pltpu.sync_copy(acc.at[pl.ds(s * npt, npt)],
                        out_r.at[pl.ds(c * n + s * npt, npt)])
        if counts:
            pltpu.sync_copy(hist, hout_r.at[c * NS + s])

    out_type = [jax.ShapeDtypeStruct((2 * n, w), jnp.float32)]
    scratch = (
        [pltpu.VMEM((CP, KP), jnp.int32),
         pltpu.VMEM((CP, KP), jnp.int32)]
        + [pltpu.VMEM((K, w), jnp.float32)] * NB
        + [pltpu.VMEM((KS,), jnp.int32)] * (2 * NB)
        + [pltpu.VMEM_SHARED((n, w), jnp.float32)]
    )
    args = [table, gidx, sidx, zeros_w]
    if counts:
        out_type.append(jax.ShapeDtypeStruct((2 * NS, n), jnp.float32))
        scratch.append(pltpu.VMEM((n,), jnp.float32))
        args.append(zeros_n)
    scratch += [pltpu.SemaphoreType.DMA] * (2 * NB)

    f = pl.kernel(
        body,
        out_type=tuple(out_type),
        mesh=plsc.VectorSubcoreMesh(core_axis_name="c", subcore_axis_name="s"),
        scratch_types=scratch,
        compiler_params=pltpu.CompilerParams(use_tc_tiling_on_sc=False,
                                             needs_layout_passes=False),
    )
    return f(*args)


def _pack_idx(idx2d):
    rows = idx2d.shape[0]
    padded = jnp.concatenate(
        [idx2d, jnp.zeros((rows, 16), jnp.int32)], axis=1)
    quads = padded.reshape(rows, 3, 2, 16)
    return (quads[:, :, 0, :] | (quads[:, :, 1, :] << 16)).reshape(rows, KP)


def _tc_dense_body(sum_ref, hist_ref, x_ref, wl_ref, wr_ref, b_ref, g_ref,
                   scnt_ref):
    agg = jnp.concatenate([sum_ref[0], sum_ref[1]], axis=1)
    cnt = jnp.sum(hist_ref[0], axis=0)[:, None]
    mean = agg / jnp.maximum(cnt, 1.0)
    h = jnp.dot(mean, wl_ref[:], preferred_element_type=jnp.float32)
    h += jnp.dot(x_ref[:], wr_ref[:], preferred_element_type=jnp.float32)
    h = jnp.maximum(h + b_ref[:], 0.0)
    g_ref[0] = h
    g_ref[1] = h * h
    scnt_ref[...] = jnp.sum(hist_ref[1], axis=0)[:, None]


def _tc_final_body(acc_ref, scnt_ref, g_ref, gg_ref):
    s1 = acc_ref[0]
    s2 = acc_ref[1]
    scnt = scnt_ref[...]
    h = g_ref[0]
    h2 = g_ref[1]
    num = scnt * h2 - 2.0 * h * s1 + s2
    gg_ref[:] = jnp.tanh(num / jnp.maximum(scnt, 1.0))


def kernel(X, edge_index, W_l, W_r, b):
    N, D = X.shape
    E = edge_index.shape[1]
    assert D == 128 and E % (K * NC * NS) == 0 and N % NS == 0

    src = edge_index[0]
    dst = edge_index[1]
    src2d = src.reshape(E // K, K)
    dst2d = dst.reshape(E // K, K)
    srcx2d = _pack_idx(jnp.concatenate([src2d, src2d + N], axis=0))
    dstx2d = _pack_idx(jnp.concatenate([dst2d, dst2d + N], axis=0))
    src2d = _pack_idx(src2d)
    dst2d = _pack_idx(dst2d)

    xcols = jnp.concatenate([X[:, :64], X[:, 64:]], axis=0)
    zeros64 = jnp.zeros((N, 64), jnp.float32)
    zeros128 = jnp.zeros((N, 128), jnp.float32)
    zeros_n = jnp.zeros((N,), jnp.float32)

    sums, hists = _sc_pass(xcols, srcx2d, dst2d, zeros64, zeros_n,
                           n=N, w=64, counts=True, nbuf=3)

    g, scnt = pl.pallas_call(
        _tc_dense_body,
        out_shape=(jax.ShapeDtypeStruct((2, N, D), jnp.float32),
                   jax.ShapeDtypeStruct((N, 1), jnp.float32)),
    )(sums.reshape(2, N, 64), hists.reshape(2, NS, N), X, W_l, W_r,
      b.reshape(1, D))

    (acc3,) = _sc_pass(g.reshape(2 * N, D), dstx2d, src2d, zeros128,
                       n=N, w=D, counts=False, nbuf=3, nphase=2)

    gg = pl.pallas_call(
        _tc_final_body,
        out_shape=jax.ShapeDtypeStruct((N, D), jnp.float32),
    )(acc3.reshape(2, N, D), scnt, g)
    return gg


# speedup vs baseline: 11.1602x; 1.0755x over previous
import jax
import jax.numpy as jnp
from jax import lax
from jax.experimental import pallas as pl
from jax.experimental.pallas import tpu as pltpu
from jax.experimental.pallas import tpu_sc as plsc

NC = 2
NS = 16
K = 80
KP = (K + 31) // 32 * 16


def _unpack_chunk(buf, j, stage):
    vecs = []
    for blk in range(KP // 16):
        word = buf[j, pl.ds(blk * 16, 16)]
        lo = word & 0xFFFF
        hi = word >> 16
        stage[pl.ds(blk * 32, 16)] = lo
        stage[pl.ds(blk * 32 + 16, 16)] = hi
        vecs += [lo, hi]
    return vecs[:K // 16]


def _sc_pass(table, gidx, sidx, *, n, w, counts, nphase=1):
    R = gidx.shape[0] // NC
    C = R // NS
    CP = C // nphase
    npt = n // NS
    KS = 2 * KP
    NB = 3
    assert C % nphase == 0

    def body(*refs):
        if counts:
            (table_r, gidx_r, sidx_r, out_r, hout_r, gbuf, sbuf,
             *rest) = refs
        else:
            (table_r, gidx_r, sidx_r, out_r, gbuf, sbuf, *rest) = refs
        rows = rest[0:NB]
        gi = rest[NB:2 * NB]
        si = rest[2 * NB:3 * NB]
        acc = rest[3 * NB]
        p = 3 * NB + 1
        if counts:
            hist = rest[p]
            p += 1
        gsem = rest[p:p + NB]
        ssem = rest[p + NB:p + 2 * NB]
        c = lax.axis_index("c")
        s = lax.axis_index("s")

        zv = jnp.zeros((16,), jnp.float32)

        @pl.loop(0, K)
        def _(r):
            for q in range(w // 16):
                rows[0][r, pl.ds(q * 16, 16)] = zv

        base = s * npt
        for off in range(0, npt - K + 1, K):
            pltpu.sync_copy(rows[0], acc.at[pl.ds(base + off, K)])
        tail = npt % K
        if tail:
            pltpu.sync_copy(rows[0].at[pl.ds(0, tail)],
                            acc.at[pl.ds(base + npt - tail, tail)])
        if counts:
            @pl.loop(0, n // 16)
            def _(r):
                hist[pl.ds(r * 16, 16)] = zv
        plsc.subcore_barrier()

        def slot(j, b):
            pltpu.make_async_copy(table_r.at[gi[b].at[pl.ds(0, K)]],
                                  rows[b], gsem[b]).wait()
            svecs = _unpack_chunk(sbuf, j, si[b])
            pltpu.async_copy(rows[b], acc.at[si[b].at[pl.ds(0, K)]],
                             ssem[b], add=True)
            if counts:
                ones = jnp.ones((16,), jnp.float32)

                @pl.when(c == 0)
                def _hist0():
                    for a in svecs:
                        plsc.addupdate_scatter(hist, [a], ones)

                @pl.when(c == 1)
                def _hist1():
                    for t in range(K // 16):
                        v = gi[b][pl.ds(t * 16, 16)] >> 1
                        plsc.addupdate_scatter(hist, [v], ones)
            bp = (b + 2) % NB

            @pl.when(j + 2 >= NB)
            def _drain():
                pltpu.make_async_copy(
                    rows[bp], acc.at[si[bp].at[pl.ds(0, K)]],
                    ssem[bp]).wait()

            @pl.when(j + 2 < CP)
            def _issue():
                _unpack_chunk(gbuf, j + 2, gi[bp])
                pltpu.async_copy(table_r.at[gi[bp].at[pl.ds(0, K)]],
                                 rows[bp], gsem[bp])

        CB = CP - CP % NB
        for ph in range(nphase):
            pltpu.sync_copy(
                gidx_r.at[pl.ds(c * R + s * C + ph * CP, CP)], gbuf)
            pltpu.sync_copy(sidx_r.at[pl.ds(s * C + ph * CP, CP)], sbuf)
            for b in range(2):
                _unpack_chunk(gbuf, b, gi[b])
                pltpu.async_copy(table_r.at[gi[b].at[pl.ds(0, K)]],
                                 rows[b], gsem[b])

            @pl.loop(0, CB, step=NB)
            def _(i):
                for off in range(NB):
                    slot(i + off, off)

            for j in range(CB, CP):
                slot(jnp.int32(j), j % NB)
            for j in range(CP + 2 - NB, CP):
                bl = j % NB
                pltpu.make_async_copy(rows[bl],
                                      acc.at[si[bl].at[pl.ds(0, K)]],
                                      ssem[bl]).wait()

        plsc.subcore_barrier()
        pltpu.sync_copy(acc.at[pl.ds(s * npt, npt)],
                        out_r.at[pl.ds(c * n + s * npt, npt)])
        if counts:
            pltpu.sync_copy(hist, hout_r.at[c * NS + s])

    out_type = [jax.ShapeDtypeStruct((2 * n, w), jnp.float32)]
    scratch = (
        [pltpu.VMEM((CP, KP), jnp.int32),
         pltpu.VMEM((CP, KP), jnp.int32)]
        + [pltpu.VMEM((K, w), jnp.float32)] * NB
        + [pltpu.VMEM((KS,), jnp.int32)] * (2 * NB)
        + [pltpu.VMEM_SHARED((n, w), jnp.float32)]
    )
    if counts:
        out_type.append(jax.ShapeDtypeStruct((2 * NS, n), jnp.float32))
        scratch.append(pltpu.VMEM((n,), jnp.float32))
    scratch += [pltpu.SemaphoreType.DMA] * (2 * NB)

    f = pl.kernel(
        body,
        out_type=tuple(out_type),
        mesh=plsc.VectorSubcoreMesh(core_axis_name="c", subcore_axis_name="s"),
        scratch_types=scratch,
        compiler_params=pltpu.CompilerParams(use_tc_tiling_on_sc=False,
                                             needs_layout_passes=False),
    )
    return f(table, gidx, sidx)


def _pack_idx(idx2d):
    rows = idx2d.shape[0]
    padded = jnp.concatenate(
        [idx2d, jnp.zeros((rows, 16), jnp.int32)], axis=1)
    quads = padded.reshape(rows, KP // 16, 2, 16)
    return (quads[:, :, 0, :] | (quads[:, :, 1, :] << 16)).reshape(rows, KP)


def _tc_dense_body(sum_ref, hist_ref, x_ref, wl_ref, wr_ref, b_ref, g_ref):
    agg = jnp.concatenate([sum_ref[0], sum_ref[1]], axis=1)
    cnt = jnp.sum(hist_ref[0], axis=0)[:, None]
    mean = agg / jnp.maximum(cnt, 1.0)
    h = jnp.dot(mean, wl_ref[:], preferred_element_type=jnp.float32)
    h += jnp.dot(x_ref[:], wr_ref[:], preferred_element_type=jnp.float32)
    h = jnp.maximum(h + b_ref[:], 0.0)
    g_ref[0] = h
    g_ref[1] = h * h


def _tc_final_body(acc_ref, hist_ref, g_ref, gg_ref):
    s1 = acc_ref[0]
    s2 = acc_ref[1]
    scnt = jnp.sum(hist_ref[1], axis=0)[:, None]
    h = g_ref[0]
    h2 = g_ref[1]
    num = scnt * h2 - 2.0 * h * s1 + s2
    gg_ref[:] = jnp.tanh(num / jnp.maximum(scnt, 1.0))


def kernel(X, edge_index, W_l, W_r, b):
    N, D = X.shape
    E = edge_index.shape[1]
    assert D == 128 and E % (K * NC * NS) == 0 and N % NS == 0

    src = edge_index[0]
    dst = edge_index[1]
    src2d = src.reshape(E // K, K)
    dst2d = dst.reshape(E // K, K)
    srcx2d = _pack_idx(jnp.concatenate([2 * src2d, 2 * src2d + 1], axis=0))
    dstx2d = _pack_idx(jnp.concatenate([dst2d, dst2d + N], axis=0))
    src2dp = _pack_idx(src2d)
    dst2dp = _pack_idx(dst2d)

    sums, hists = _sc_pass(X.reshape(2 * N, 64), srcx2d, dst2dp,
                           n=N, w=64, counts=True)
    hists = hists.reshape(2, NS, N)

    g = pl.pallas_call(
        _tc_dense_body,
        out_shape=jax.ShapeDtypeStruct((2, N, D), jnp.float32),
    )(sums.reshape(2, N, 64), hists, X, W_l, W_r, b.reshape(1, D))

    (acc3,) = _sc_pass(g.reshape(2 * N, D), dstx2d, src2dp,
                       n=N, w=D, counts=False, nphase=2)

    gg = pl.pallas_call(
        _tc_final_body,
        out_shape=jax.ShapeDtypeStruct((N, D), jnp.float32),
    )(acc3.reshape(2, N, D), hists, g)
    return gg


# speedup vs baseline: 12.1502x; 1.0887x over previous
import jax
import jax.numpy as jnp
from jax import lax
from jax.experimental import pallas as pl
from jax.experimental.pallas import tpu as pltpu
from jax.experimental.pallas import tpu_sc as plsc

NC = 2
NS = 16
K = 80
KP = (K + 31) // 32 * 16


def _unpack_chunk(buf, j, stage):
    vecs = []
    for blk in range(KP // 16):
        word = buf[j, pl.ds(blk * 16, 16)]
        lo = word & 0xFFFF
        hi = word >> 16
        stage[pl.ds(blk * 32, 16)] = lo
        stage[pl.ds(blk * 32 + 16, 16)] = hi
        vecs += [lo, hi]
    return vecs[:K // 16]


def _sc_pass(table, gidx, sidx, *, n, w, counts, nphase=1,
             edge_split=False):
    R = gidx.shape[0] // NC
    C = R // NS
    CP = C // nphase
    npt = n // NS
    KS = 2 * KP
    NB = 3
    assert C % nphase == 0

    def body(*refs):
        if counts:
            (table_r, gidx_r, sidx_r, out_r, hout_r, gbuf, sbuf,
             *rest) = refs
        else:
            (table_r, gidx_r, sidx_r, out_r, gbuf, sbuf, *rest) = refs
        rows = rest[0:NB]
        gi = rest[NB:2 * NB]
        si = rest[2 * NB:3 * NB]
        acc = rest[3 * NB]
        p = 3 * NB + 1
        if counts:
            hist = rest[p]
            p += 1
        gsem = rest[p:p + NB]
        ssem = rest[p + NB:p + 2 * NB]
        c = lax.axis_index("c")
        s = lax.axis_index("s")

        zv = jnp.zeros((16,), jnp.float32)

        @pl.loop(0, K)
        def _(r):
            for q in range(w // 16):
                rows[0][r, pl.ds(q * 16, 16)] = zv

        base = s * npt
        for off in range(0, npt - K + 1, K):
            pltpu.sync_copy(rows[0], acc.at[pl.ds(base + off, K)])
        tail = npt % K
        if tail:
            pltpu.sync_copy(rows[0].at[pl.ds(0, tail)],
                            acc.at[pl.ds(base + npt - tail, tail)])
        if counts:
            @pl.loop(0, n // 16)
            def _(r):
                hist[pl.ds(r * 16, 16)] = zv
        plsc.subcore_barrier()

        def slot(j, b):
            pltpu.make_async_copy(table_r.at[gi[b].at[pl.ds(0, K)]],
                                  rows[b], gsem[b]).wait()
            svecs = _unpack_chunk(sbuf, j, si[b])
            pltpu.async_copy(rows[b], acc.at[si[b].at[pl.ds(0, K)]],
                             ssem[b], add=True)
            if counts:
                ones = jnp.ones((16,), jnp.float32)

                def _hist():
                    for a in svecs:
                        plsc.addupdate_scatter(hist, [a], ones)

                if counts == 2:
                    _hist()
                else:
                    pl.when(c == 0)(_hist)
            bp = (b + 2) % NB

            @pl.when(j + 2 >= NB)
            def _drain():
                pltpu.make_async_copy(
                    rows[bp], acc.at[si[bp].at[pl.ds(0, K)]],
                    ssem[bp]).wait()

            @pl.when(j + 2 < CP)
            def _issue():
                _unpack_chunk(gbuf, j + 2, gi[bp])
                pltpu.async_copy(table_r.at[gi[bp].at[pl.ds(0, K)]],
                                 rows[bp], gsem[bp])

        CB = CP - CP % NB
        sbase = c * R + s * C if edge_split else s * C
        for ph in range(nphase):
            pltpu.sync_copy(
                gidx_r.at[pl.ds(c * R + s * C + ph * CP, CP)], gbuf)
            pltpu.sync_copy(sidx_r.at[pl.ds(sbase + ph * CP, CP)], sbuf)
            for b in range(2):
                _unpack_chunk(gbuf, b, gi[b])
                pltpu.async_copy(table_r.at[gi[b].at[pl.ds(0, K)]],
                                 rows[b], gsem[b])

            @pl.loop(0, CB, step=NB)
            def _(i):
                for off in range(NB):
                    slot(i + off, off)

            for j in range(CB, CP):
                slot(jnp.int32(j), j % NB)
            for j in range(CP + 2 - NB, CP):
                bl = j % NB
                pltpu.make_async_copy(rows[bl],
                                      acc.at[si[bl].at[pl.ds(0, K)]],
                                      ssem[bl]).wait()

        plsc.subcore_barrier()
        pltpu.sync_copy(acc.at[pl.ds(s * npt, npt)],
                        out_r.at[pl.ds(c * n + s * npt, npt)])
        if counts == 2:
            pltpu.sync_copy(hist, hout_r.at[c * NS + s])
        elif counts == 1:
            @pl.when(c == 0)
            def _hw():
                pltpu.sync_copy(hist, hout_r.at[s])

    out_type = [jax.ShapeDtypeStruct((2 * n, w), jnp.float32)]
    scratch = (
        [pltpu.VMEM((CP, KP), jnp.int32),
         pltpu.VMEM((CP, KP), jnp.int32)]
        + [pltpu.VMEM((K, w), jnp.float32)] * NB
        + [pltpu.VMEM((KS,), jnp.int32)] * (2 * NB)
        + [pltpu.VMEM_SHARED((n, w), jnp.float32)]
    )
    if counts:
        out_type.append(
            jax.ShapeDtypeStruct((NS * counts, n), jnp.float32))
        scratch.append(pltpu.VMEM((n,), jnp.float32))
    scratch += [pltpu.SemaphoreType.DMA] * (2 * NB)

    f = pl.kernel(
        body,
        out_type=tuple(out_type),
        mesh=plsc.VectorSubcoreMesh(core_axis_name="c", subcore_axis_name="s"),
        scratch_types=scratch,
        compiler_params=pltpu.CompilerParams(use_tc_tiling_on_sc=False,
                                             needs_layout_passes=False),
    )
    return f(table, gidx, sidx)


def _pack_idx(idx2d):
    rows = idx2d.shape[0]
    padded = jnp.concatenate(
        [idx2d, jnp.zeros((rows, 16), jnp.int32)], axis=1)
    quads = padded.reshape(rows, KP // 16, 2, 16)
    return (quads[:, :, 0, :] | (quads[:, :, 1, :] << 16)).reshape(rows, KP)


def _tc_dense_body(sum_ref, hist_ref, x_ref, wl_ref, wr_ref, b_ref, g_ref):
    agg = sum_ref[0] + sum_ref[1]
    cnt = jnp.sum(hist_ref[...], axis=0)[:, None]
    mean = agg / jnp.maximum(cnt, 1.0)
    h = jnp.dot(mean, wl_ref[:], preferred_element_type=jnp.float32)
    h += jnp.dot(x_ref[:], wr_ref[:], preferred_element_type=jnp.float32)
    h = jnp.maximum(h + b_ref[:], 0.0)
    g_ref[0] = h
    g_ref[1] = h * h


def _tc_final_body(acc_ref, hist_ref, g_ref, gg_ref):
    s1 = acc_ref[0]
    s2 = acc_ref[1]
    scnt = jnp.sum(hist_ref[...], axis=0)[:, None]
    h = g_ref[0]
    h2 = g_ref[1]
    num = scnt * h2 - 2.0 * h * s1 + s2
    gg_ref[:] = jnp.tanh(num / jnp.maximum(scnt, 1.0))


def kernel(X, edge_index, W_l, W_r, b):
    N, D = X.shape
    E = edge_index.shape[1]
    assert D == 128 and E % (K * NC * NS) == 0 and N % NS == 0

    src = edge_index[0]
    dst = edge_index[1]
    src2d = src.reshape(E // K, K)
    dst2d = dst.reshape(E // K, K)
    dstx2d = _pack_idx(jnp.concatenate([dst2d, dst2d + N], axis=0))
    src2dp = _pack_idx(src2d)
    dst2dp = _pack_idx(dst2d)

    sums, hists_d = _sc_pass(X, src2dp, dst2dp, n=N, w=D, counts=2,
                             nphase=5, edge_split=True)

    g = pl.pallas_call(
        _tc_dense_body,
        out_shape=jax.ShapeDtypeStruct((2, N, D), jnp.float32),
    )(sums.reshape(2, N, D), hists_d, X, W_l, W_r, b.reshape(1, D))

    acc3, hists_s = _sc_pass(g.reshape(2 * N, D), dstx2d, src2dp,
                             n=N, w=D, counts=1, nphase=5)

    gg = pl.pallas_call(
        _tc_final_body,
        out_shape=jax.ShapeDtypeStruct((N, D), jnp.float32),
    )(acc3.reshape(2, N, D), hists_s, g)
    return gg
